# baseline (speedup 1.0000x reference)
import functools

import jax
import jax.numpy as jnp
from jax import lax
from jax.experimental import pallas as pl
from jax.experimental.pallas import tpu as pltpu
from jax.experimental.pallas import tpu_sc as plsc

_NUM_FEATURES = 26
_CATEGORY_NUM = 32
_BATCH = 16384

_L = 16
_NW = 32
_TOTAL = _BATCH * _NUM_FEATURES
_PER_W = _TOTAL // _NW
_VECS = _PER_W // _L
_TABLE = _NUM_FEATURES * _CATEGORY_NUM
_BLK = 128
_CHUNK_BLKS = 8
_CHUNKS = _PER_W // (_BLK * _CHUNK_BLKS)


def _body(idx_hbm, bias_hbm, mean_hbm, norm_hbm, out_hbm,
          idx_v, out_v, bias_v, table_v, mean_v, norm_v, sem, osem):
    wid = lax.axis_index("s") * 2 + lax.axis_index("c")
    base = wid * _PER_W

    idx_cp = pltpu.async_copy(idx_hbm.at[pl.ds(base, _PER_W)], idx_v, sem)

    pltpu.sync_copy(bias_hbm, bias_v)
    pltpu.sync_copy(mean_hbm, mean_v.at[pl.ds(0, _NUM_FEATURES)])
    pltpu.sync_copy(norm_hbm, norm_v.at[pl.ds(0, _NUM_FEATURES)])

    lanes = jax.lax.iota(jnp.int32, _L)

    def fold(j, _):
        p = lanes + j * _L
        f = jax.lax.shift_right_logical(p, 5)
        m = plsc.load_gather(mean_v, [f])
        n = plsc.load_gather(norm_v, [f])
        b = bias_v[pl.ds(j * _L, _L)]
        table_v[pl.ds(j * _L, _L)] = (b - m) / n
        return _

    lax.fori_loop(0, _TABLE // _L, fold, 0, unroll=4)

    idx_cp.wait()

    cps = []
    for c in range(_CHUNKS):
        gc = wid * _CHUNKS + c
        f = jax.lax.shift_right_logical(gc, 4)
        foff = jnp.broadcast_to(jax.lax.shift_left(f, 5), (_L,))
        cbase = c * _CHUNK_BLKS * _BLK

        @plsc.parallel_loop(0, _CHUNK_BLKS * _BLK // _L, 1, unroll=16)
        def _(j, cbase=cbase, foff=foff):
            off = cbase + j * _L
            g = idx_v[pl.ds(off, _L)] + foff
            out_v[jax.lax.shift_right_logical(off, 7),
                  pl.ds(jax.lax.rem(off, _BLK), _L)] = \
                plsc.load_gather(table_v, [g])

        l0 = jax.lax.shift_left(jax.lax.rem(gc, 16), 3)
        cps.append(pltpu.async_copy(
            out_v.at[pl.ds(c * _CHUNK_BLKS, _CHUNK_BLKS)],
            out_hbm.at[jax.lax.shift_right_logical(f, 3),
                       pl.ds(l0, _CHUNK_BLKS),
                       jax.lax.rem(f, 8)],
            osem))
    for cp in cps:
        cp.wait()


@jax.jit
def _run(idx_flat, bias_flat, mean, norm):
    mesh = plsc.VectorSubcoreMesh(core_axis_name="c", subcore_axis_name="s")
    kern = functools.partial(
        pl.kernel,
        mesh=mesh,
        compiler_params=pltpu.CompilerParams(needs_layout_passes=False),
        out_type=jax.ShapeDtypeStruct((4, _BLK, 8, _BLK), jnp.float32),
        scratch_types=[
            pltpu.VMEM((_PER_W,), jnp.int32),
            pltpu.VMEM((_PER_W // _BLK, _BLK), jnp.float32),
            pltpu.VMEM((_TABLE,), jnp.float32),
            pltpu.VMEM((_TABLE,), jnp.float32),
            pltpu.VMEM((128,), jnp.float32),
            pltpu.VMEM((128,), jnp.float32),
            pltpu.SemaphoreType.DMA,
            pltpu.SemaphoreType.DMA,
        ],
    )(_body)
    return kern(idx_flat, bias_flat, mean, norm)


def kernel(inputs, bias, moving_mean, moving_norm):
    idx_flat = inputs.T.reshape(_TOTAL)
    bias_flat = bias.reshape(_TABLE)
    out4d = _run(idx_flat, bias_flat, moving_mean, moving_norm)
    out = out4d.transpose(1, 3, 0, 2).reshape(_BATCH, 32)[:, :_NUM_FEATURES]
    return out


# speedup vs baseline: 1.0142x; 1.0142x over previous
import functools

import jax
import jax.numpy as jnp
from jax import lax
from jax.experimental import pallas as pl
from jax.experimental.pallas import tpu as pltpu
from jax.experimental.pallas import tpu_sc as plsc

_NUM_FEATURES = 26
_CATEGORY_NUM = 32
_BATCH = 16384

_L = 16
_NW = 32
_TOTAL = _BATCH * _NUM_FEATURES
_PER_W = _TOTAL // _NW
_VECS = _PER_W // _L
_TABLE = _NUM_FEATURES * _CATEGORY_NUM
_BLK = 128
_CHUNK_BLKS = 8
_CHUNKS = _PER_W // (_BLK * _CHUNK_BLKS)


def _body(idx_hbm, bias_hbm, mean_hbm, norm_hbm, out_hbm,
          idx_v, out_v, bias_v, table_v, mean_v, norm_v, sem, osem):
    wid = lax.axis_index("s") * 2 + lax.axis_index("c")
    base = wid * _PER_W

    idx_cp = pltpu.async_copy(idx_hbm.at[pl.ds(base, _PER_W)], idx_v, sem)

    pltpu.sync_copy(bias_hbm, bias_v)
    pltpu.sync_copy(mean_hbm, mean_v.at[pl.ds(0, _NUM_FEATURES)])
    pltpu.sync_copy(norm_hbm, norm_v.at[pl.ds(0, _NUM_FEATURES)])

    lanes = jax.lax.iota(jnp.int32, _L)

    def fold(j, _):
        p = lanes + j * _L
        f = jax.lax.shift_right_logical(p, 5)
        m = plsc.load_gather(mean_v, [f])
        n = plsc.load_gather(norm_v, [f])
        b = bias_v[pl.ds(j * _L, _L)]
        table_v[pl.ds(j * _L, _L)] = (b - m) / n
        return _

    lax.fori_loop(0, _TABLE // _L, fold, 0, unroll=4)

    idx_cp.wait()

    cps = []
    for c in range(_CHUNKS):
        gc = wid * _CHUNKS + c
        f = jax.lax.shift_right_logical(gc, 4)
        foff = jnp.broadcast_to(jax.lax.shift_left(f, 5), (_L,))
        cbase = c * _CHUNK_BLKS * _BLK

        @plsc.parallel_loop(0, _CHUNK_BLKS * _BLK // _L, 1, unroll=4)
        def _(j, cbase=cbase, foff=foff):
            off = cbase + j * _L
            g = idx_v[pl.ds(off, _L)] + foff
            out_v[jax.lax.shift_right_logical(off, 7),
                  pl.ds(jax.lax.rem(off, _BLK), _L)] = \
                plsc.load_gather(table_v, [g])

        l0 = jax.lax.shift_left(jax.lax.rem(gc, 16), 3)
        cps.append(pltpu.async_copy(
            out_v.at[pl.ds(c * _CHUNK_BLKS, _CHUNK_BLKS)],
            out_hbm.at[jax.lax.shift_right_logical(f, 3),
                       pl.ds(l0, _CHUNK_BLKS),
                       jax.lax.rem(f, 8)],
            osem))
    for cp in cps:
        cp.wait()


@jax.jit
def _run(idx_flat, bias_flat, mean, norm):
    mesh = plsc.VectorSubcoreMesh(core_axis_name="c", subcore_axis_name="s")
    kern = functools.partial(
        pl.kernel,
        mesh=mesh,
        compiler_params=pltpu.CompilerParams(needs_layout_passes=False),
        out_type=jax.ShapeDtypeStruct((4, _BLK, 8, _BLK), jnp.float32),
        scratch_types=[
            pltpu.VMEM((_PER_W,), jnp.int32),
            pltpu.VMEM((_PER_W // _BLK, _BLK), jnp.float32),
            pltpu.VMEM((_TABLE,), jnp.float32),
            pltpu.VMEM((_TABLE,), jnp.float32),
            pltpu.VMEM((128,), jnp.float32),
            pltpu.VMEM((128,), jnp.float32),
            pltpu.SemaphoreType.DMA,
            pltpu.SemaphoreType.DMA,
        ],
    )(_body)
    return kern(idx_flat, bias_flat, mean, norm)


def kernel(inputs, bias, moving_mean, moving_norm):
    idx_flat = inputs.T.reshape(_TOTAL)
    bias_flat = bias.reshape(_TABLE)
    out4d = _run(idx_flat, bias_flat, moving_mean, moving_norm)
    out = out4d.transpose(1, 3, 0, 2).reshape(_BATCH, 32)[:, :_NUM_FEATURES]
    return out


# speedup vs baseline: 1.1031x; 1.0877x over previous
import functools

import jax
import jax.numpy as jnp
from jax import lax
from jax.experimental import pallas as pl
from jax.experimental.pallas import tpu as pltpu
from jax.experimental.pallas import tpu_sc as plsc

_NUM_FEATURES = 26
_CATEGORY_NUM = 32
_BATCH = 16384

_L = 16
_NW = 32
_TOTAL = _BATCH * _NUM_FEATURES
_PER_W = _TOTAL // _NW
_VECS = _PER_W // _L
_TABLE = _NUM_FEATURES * _CATEGORY_NUM
_BLK = 128
_BW = _BATCH // _NW
_LB = _BW // _BLK


def _body(idx_hbm, bias_hbm, mean_hbm, norm_hbm, out_hbm,
          idx_v, out_v, bias_v, table_v, mean_v, norm_v, sem):
    wid = lax.axis_index("s") * 2 + lax.axis_index("c")
    b0 = wid * _BW

    idx_cp = pltpu.async_copy(
        idx_hbm.at[:, pl.ds(b0, _BW)], idx_v, sem)

    pltpu.sync_copy(bias_hbm, bias_v)
    pltpu.sync_copy(mean_hbm, mean_v.at[pl.ds(0, _NUM_FEATURES)])
    pltpu.sync_copy(norm_hbm, norm_v.at[pl.ds(0, _NUM_FEATURES)])

    lanes = jax.lax.iota(jnp.int32, _L)

    def fold(j, _):
        p = lanes + j * _L
        f = jax.lax.shift_right_logical(p, 5)
        m = plsc.load_gather(mean_v, [f])
        n = plsc.load_gather(norm_v, [f])
        b = bias_v[pl.ds(j * _L, _L)]
        table_v[pl.ds(j * _L, _L)] = (b - m) / n
        return _

    lax.fori_loop(0, _TABLE // _L, fold, 0, unroll=4)

    idx_cp.wait()

    @plsc.parallel_loop(0, _VECS, 1, unroll=8)
    def _(j):
        f = jax.lax.shift_right_logical(j, 5)
        cv = jax.lax.rem(j, 32)
        g = idx_v[f, pl.ds(cv * _L, _L)] + jax.lax.shift_left(f, 5)
        out_v[jax.lax.shift_right_logical(f, 3),
              jax.lax.shift_right_logical(cv, 3),
              jax.lax.rem(f, 8),
              pl.ds(jax.lax.rem(cv, 8) * _L, _L)] = \
            plsc.load_gather(table_v, [g])

    pltpu.sync_copy(out_v, out_hbm.at[:, pl.ds(wid * _LB, _LB)])


@jax.jit
def _run(idx_2d, bias_flat, mean, norm):
    mesh = plsc.VectorSubcoreMesh(core_axis_name="c", subcore_axis_name="s")
    kern = functools.partial(
        pl.kernel,
        mesh=mesh,
        compiler_params=pltpu.CompilerParams(
            needs_layout_passes=False, use_tc_tiling_on_sc=True),
        out_type=jax.ShapeDtypeStruct((4, _BLK, 8, _BLK), jnp.float32),
        scratch_types=[
            pltpu.VMEM((_NUM_FEATURES, _BW), jnp.int32),
            pltpu.VMEM((4, _LB, 8, _BLK), jnp.float32),
            pltpu.VMEM((_TABLE,), jnp.float32),
            pltpu.VMEM((_TABLE,), jnp.float32),
            pltpu.VMEM((128,), jnp.float32),
            pltpu.VMEM((128,), jnp.float32),
            pltpu.SemaphoreType.DMA,
        ],
    )(_body)
    return kern(idx_2d, bias_flat, mean, norm)


def kernel(inputs, bias, moving_mean, moving_norm):
    idx_2d = inputs.T
    bias_flat = bias.reshape(_TABLE)
    out4d = _run(idx_2d, bias_flat, moving_mean, moving_norm)
    out = out4d.transpose(1, 3, 0, 2).reshape(_BATCH, 32)[:, :_NUM_FEATURES]
    return out
